# baseline (speedup 1.0000x reference)
import jax
import jax.numpy as jnp
from jax import lax
from jax.experimental import pallas as pl
from jax.experimental.pallas import tpu as pltpu
from jax.experimental.pallas import tpu_sc as plsc

K = 16
NSLAB = 64
GATHERS = 8
WORDROWS = 256


def _sc_body(pred_hbm, words_hbm, cls_hbm, out_hbm,
             cls_v, idx_v, mask_v, eb0, eb1, part_v, sem0, sem1):
    c = lax.axis_index("c")
    s = lax.axis_index("s")
    wid = s * 2 + c

    pltpu.sync_copy(cls_hbm, cls_v)
    pltpu.sync_copy(words_hbm.at[pl.ds(wid * WORDROWS, WORDROWS)], mask_v)

    lane = lax.iota(jnp.int32, 16)
    cls_splat = plsc.load_gather(cls_v, [jnp.full((16,), s, jnp.int32)])
    base = cls_splat * NSLAB + c * (NSLAB // 2) + lane
    plsc.store_scatter(idx_v, [lane // 4, lane % 4], base)
    plsc.store_scatter(idx_v, [lane // 4 + 4, lane % 4], base + 16)

    bufs = (eb0, eb1)
    sems = (sem0, sem1)

    def issue(g):
        return pltpu.async_copy(pred_hbm.at[idx_v.at[g]], bufs[g % 2], sems[g % 2])

    zf = jnp.zeros((16,), jnp.float32)
    acc = (zf, zf, jnp.zeros((16,), jnp.uint32))
    h = issue(0)
    for g in range(GATHERS):
        h.wait()
        if g + 1 < GATHERS:
            h = issue(g + 1)
        buf = bufs[g % 2]

        def row_body(t, carry, g=g, buf=buf):
            sA, sB, cn = carry
            w = mask_v[g * 32 + t]
            i2 = t // 8
            r = t % 8
            for j in range(32):
                e = buf[i2, r, pl.ds(j * 16, 16)]
                msk = (w & jnp.uint32(1 << j)) != jnp.uint32(0)
                em = jnp.where(msk, e, 0.0)
                sA = sA + em
                sB = sB + em * em
            x = w - ((w >> jnp.uint32(1)) & jnp.uint32(0x55555555))
            x = (x & jnp.uint32(0x33333333)) + ((x >> jnp.uint32(2)) & jnp.uint32(0x33333333))
            x = (x + (x >> jnp.uint32(4))) & jnp.uint32(0x0F0F0F0F)
            cn = cn + ((x * jnp.uint32(0x01010101)) >> jnp.uint32(24))
            return (sA, sB, cn)

        acc = lax.fori_loop(0, 32, row_body, acc)

    part_v[0] = acc[0]
    part_v[1] = acc[1]
    part_v[2] = acc[2].astype(jnp.float32)
    pltpu.sync_copy(part_v, out_hbm.at[wid])


_sc_call = pl.kernel(
    _sc_body,
    out_type=jax.ShapeDtypeStruct((32, 3, 16), jnp.float32),
    mesh=plsc.VectorSubcoreMesh(core_axis_name="c", subcore_axis_name="s"),
    compiler_params=pltpu.CompilerParams(needs_layout_passes=False),
    scratch_types=[
        pltpu.VMEM((16,), jnp.int32),
        pltpu.VMEM((8, 4), jnp.int32),
        pltpu.VMEM((WORDROWS, 16), jnp.uint32),
        pltpu.VMEM((4, 8, 512), jnp.float32),
        pltpu.VMEM((4, 8, 512), jnp.float32),
        pltpu.VMEM((3, 16), jnp.float32),
        pltpu.SemaphoreType.DMA,
        pltpu.SemaphoreType.DMA,
    ],
)


def _finish_body(p_ref, cls_ref, out_ref):
    x = p_ref[...]
    s = x[:, 0:16].sum(-1, keepdims=True) + x[:, 48:64].sum(-1, keepdims=True)
    s2 = x[:, 16:32].sum(-1, keepdims=True) + x[:, 64:80].sum(-1, keepdims=True)
    c = x[:, 32:48].sum(-1, keepdims=True) + x[:, 80:96].sum(-1, keepdims=True)
    safe = jnp.maximum(c, 1.0)
    means = jnp.where(c > 0, s / safe, 0.0)
    var = jnp.where(c > 0, s2 / safe - means * means, 0.0)
    row = jax.lax.broadcasted_iota(jnp.int32, (K, K), 0)
    col = jax.lax.broadcasted_iota(jnp.int32, (K, K), 1)
    eye = (row == col).astype(jnp.float32)
    mcol = jnp.broadcast_to(means, (K, K))
    mrow = (mcol * eye).sum(axis=0, keepdims=True)
    diff = mcol - mrow
    cls = cls_ref[...].astype(jnp.float32)
    ccol = (jnp.broadcast_to(cls, (K, K)) * eye).sum(axis=-1, keepdims=True)
    same = (jnp.broadcast_to(ccol, (K, K)) == cls).astype(jnp.float32)
    triu = (col > row).astype(jnp.float32)
    inter = jnp.sum(jnp.maximum(1.0 - diff * diff, 0.0) * same * triu)
    reg = jnp.mean(means * means)
    intra = jnp.mean(var)
    out_ref[...] = jnp.reshape(inter + reg + intra, (1, 1))


def _pack_mask(gt_objmask):
    m = gt_objmask.reshape(8192, 32, 16).astype(jnp.uint32)
    shifts = (jnp.uint32(1) << jnp.arange(32, dtype=jnp.uint32))[None, :, None]
    return jnp.sum(m * shifts, axis=1, dtype=jnp.uint32)


def kernel(pred_emb, gt_objmask, gt_classes):
    cls = gt_classes.astype(jnp.int32)
    pred_view = pred_emb.reshape(80 * NSLAB, 8, 512)
    words = _pack_mask(gt_objmask)
    partials = _sc_call(pred_view, words, cls)
    loss = pl.pallas_call(
        _finish_body,
        out_shape=jax.ShapeDtypeStruct((1, 1), jnp.float32),
    )(partials.reshape(K, 96), cls[None, :])
    return loss.reshape(1)


# speedup vs baseline: 1.3492x; 1.3492x over previous
import jax
import jax.numpy as jnp
from jax import lax
from jax.experimental import pallas as pl
from jax.experimental.pallas import tpu as pltpu
from jax.experimental.pallas import tpu_sc as plsc

K = 16
NSLAB = 64
GATHERS = 8
WORDROWS = 256


def _sc_body(pred_hbm, words_hbm, cls_hbm, out_hbm,
             cls_v, idx_v, mask_v, eb0, eb1, part_v, sem0, sem1):
    c = lax.axis_index("c")
    s = lax.axis_index("s")
    wid = s * 2 + c

    pltpu.sync_copy(cls_hbm, cls_v)
    pltpu.sync_copy(words_hbm.at[pl.ds(wid * WORDROWS, WORDROWS)], mask_v)

    lane = lax.iota(jnp.int32, 16)
    cls_splat = plsc.load_gather(cls_v, [jnp.full((16,), s, jnp.int32)])
    base = cls_splat * NSLAB + c * (NSLAB // 2) + lane
    plsc.store_scatter(idx_v, [lane // 4, lane % 4], base)
    plsc.store_scatter(idx_v, [lane // 4 + 4, lane % 4], base + 16)

    bufs = (eb0, eb1)
    sems = (sem0, sem1)

    def issue(g):
        return pltpu.async_copy(pred_hbm.at[idx_v.at[g]], bufs[g % 2], sems[g % 2])

    zf = jnp.zeros((16,), jnp.float32)
    acc = (zf, zf, jnp.zeros((16,), jnp.uint32))
    h = issue(0)
    for g in range(GATHERS):
        h.wait()
        if g + 1 < GATHERS:
            h = issue(g + 1)
        buf = bufs[g % 2]

        def row_body(t, carry, g=g, buf=buf):
            sA, sB, cn = carry
            w = mask_v[g * 32 + t]
            i2 = t // 8
            r = t % 8
            for j in range(32):
                e = buf[i2, r, pl.ds(j * 16, 16)]
                msk = (w & jnp.uint32(1 << j)) != jnp.uint32(0)
                em = jnp.where(msk, e, 0.0)
                sA = sA + em
                sB = sB + em * em
            x = w - ((w >> jnp.uint32(1)) & jnp.uint32(0x55555555))
            x = (x & jnp.uint32(0x33333333)) + ((x >> jnp.uint32(2)) & jnp.uint32(0x33333333))
            x = (x + (x >> jnp.uint32(4))) & jnp.uint32(0x0F0F0F0F)
            cn = cn + ((x * jnp.uint32(0x01010101)) >> jnp.uint32(24))
            return (sA, sB, cn)

        acc = lax.fori_loop(0, 32, row_body, acc)

    part_v[0] = acc[0]
    part_v[1] = acc[1]
    part_v[2] = acc[2].astype(jnp.float32)
    pltpu.sync_copy(part_v, out_hbm.at[wid])


_sc_call = pl.kernel(
    _sc_body,
    out_type=jax.ShapeDtypeStruct((32, 3, 16), jnp.float32),
    mesh=plsc.VectorSubcoreMesh(core_axis_name="c", subcore_axis_name="s"),
    compiler_params=pltpu.CompilerParams(needs_layout_passes=False),
    scratch_types=[
        pltpu.VMEM((16,), jnp.int32),
        pltpu.VMEM((8, 4), jnp.int32),
        pltpu.VMEM((WORDROWS, 16), jnp.uint32),
        pltpu.VMEM((4, 8, 512), jnp.float32),
        pltpu.VMEM((4, 8, 512), jnp.float32),
        pltpu.VMEM((3, 16), jnp.float32),
        pltpu.SemaphoreType.DMA,
        pltpu.SemaphoreType.DMA,
    ],
)


def _finish_body(p_ref, cls_ref, out_ref):
    x = p_ref[...]
    s = x[:, 0:16].sum(-1, keepdims=True) + x[:, 48:64].sum(-1, keepdims=True)
    s2 = x[:, 16:32].sum(-1, keepdims=True) + x[:, 64:80].sum(-1, keepdims=True)
    c = x[:, 32:48].sum(-1, keepdims=True) + x[:, 80:96].sum(-1, keepdims=True)
    safe = jnp.maximum(c, 1.0)
    means = jnp.where(c > 0, s / safe, 0.0)
    var = jnp.where(c > 0, s2 / safe - means * means, 0.0)
    row = jax.lax.broadcasted_iota(jnp.int32, (K, K), 0)
    col = jax.lax.broadcasted_iota(jnp.int32, (K, K), 1)
    eye = (row == col).astype(jnp.float32)
    mcol = jnp.broadcast_to(means, (K, K))
    mrow = (mcol * eye).sum(axis=0, keepdims=True)
    diff = mcol - mrow
    cls = cls_ref[...].astype(jnp.float32)
    ccol = (jnp.broadcast_to(cls, (K, K)) * eye).sum(axis=-1, keepdims=True)
    same = (jnp.broadcast_to(ccol, (K, K)) == cls).astype(jnp.float32)
    triu = (col > row).astype(jnp.float32)
    inter = jnp.sum(jnp.maximum(1.0 - diff * diff, 0.0) * same * triu)
    reg = jnp.mean(means * means)
    intra = jnp.mean(var)
    out_ref[...] = jnp.reshape(inter + reg + intra, (1, 1))


def _pack_body(mask_ref, out_ref):
    col = lax.broadcasted_iota(jnp.int32, (512, 32), 0)
    c2 = lax.broadcasted_iota(jnp.int32, (512, 32), 1)
    j = col // 16
    match = jnp.logical_and(col % 16 == c2 % 16, j // 16 == c2 // 16)
    powf = lax.bitcast_convert_type((j % 16 + 127) << 23, jnp.float32)
    proj = jnp.where(match, powf, 0.0)
    for i in range(4):
        m = mask_ref[i].astype(jnp.float32)
        r = jnp.dot(m, proj, preferred_element_type=jnp.float32)
        w = r[:, :16].astype(jnp.int32) | (r[:, 16:].astype(jnp.int32) << 16)
        out_ref[i * 512:(i + 1) * 512] = w


def _pack_mask(gt_objmask):
    words_i = pl.pallas_call(
        _pack_body,
        grid=(4,),
        in_specs=[pl.BlockSpec((4, 512, 512), lambda b: (b, 0, 0))],
        out_specs=pl.BlockSpec((2048, 16), lambda b: (b, 0)),
        out_shape=jax.ShapeDtypeStruct((8192, 16), jnp.int32),
    )(gt_objmask.view(jnp.uint8))
    return lax.bitcast_convert_type(words_i, jnp.uint32)


def kernel(pred_emb, gt_objmask, gt_classes):
    cls = gt_classes.astype(jnp.int32)
    pred_view = pred_emb.reshape(80 * NSLAB, 8, 512)
    words = _pack_mask(gt_objmask)
    partials = _sc_call(pred_view, words, cls)
    loss = pl.pallas_call(
        _finish_body,
        out_shape=jax.ShapeDtypeStruct((1, 1), jnp.float32),
    )(partials.reshape(K, 96), cls[None, :])
    return loss.reshape(1)


# speedup vs baseline: 1.5764x; 1.1684x over previous
import jax
import jax.numpy as jnp
from jax import lax
from jax.experimental import pallas as pl
from jax.experimental.pallas import tpu as pltpu
from jax.experimental.pallas import tpu_sc as plsc

K = 16
NSLAB = 64
GATHERS = 8
WORDROWS = 256


def _sc_body(pred_hbm, words_hbm, cls_hbm, out_hbm,
             cls_v, idx_v, mask_v, eb0, eb1, part_v, sem0, sem1):
    c = lax.axis_index("c")
    s = lax.axis_index("s")
    wid = s * 2 + c

    pltpu.sync_copy(cls_hbm, cls_v)
    pltpu.sync_copy(words_hbm.at[pl.ds(wid * WORDROWS, WORDROWS)], mask_v)

    lane = lax.iota(jnp.int32, 16)
    cls_splat = plsc.load_gather(cls_v, [jnp.full((16,), s, jnp.int32)])
    base = cls_splat * NSLAB + c * (NSLAB // 2) + lane
    plsc.store_scatter(idx_v, [lane // 4, lane % 4], base)
    plsc.store_scatter(idx_v, [lane // 4 + 4, lane % 4], base + 16)

    bufs = (eb0, eb1)
    sems = (sem0, sem1)

    def copy(g, b):
        return pltpu.make_async_copy(pred_hbm.at[idx_v.at[g]], bufs[b], sems[b])

    copy(0, 0).start()
    copy(1, 1).start()

    def pair_body(gi, carry):
        g0 = gi * 2
        for b in range(2):
            g = g0 + b
            copy(g, b).wait()

            def row_body(t, carry, b=b, g=g):
                sA, sB, cn = carry
                w = mask_v[g * 32 + t]
                i2 = t // 8
                r = t % 8
                buf = bufs[b]
                for j in range(32):
                    e = buf[i2, r, pl.ds(j * 16, 16)]
                    msk = (w & jnp.uint32(1 << j)) != jnp.uint32(0)
                    em = jnp.where(msk, e, 0.0)
                    sA = sA + em
                    sB = sB + em * em
                x = w - ((w >> jnp.uint32(1)) & jnp.uint32(0x55555555))
                x = (x & jnp.uint32(0x33333333)) + ((x >> jnp.uint32(2)) & jnp.uint32(0x33333333))
                x = (x + (x >> jnp.uint32(4))) & jnp.uint32(0x0F0F0F0F)
                cn = cn + ((x * jnp.uint32(0x01010101)) >> jnp.uint32(24))
                return (sA, sB, cn)

            carry = lax.fori_loop(0, 32, row_body, carry)

            @pl.when(g + 2 < GATHERS)
            def _next(g=g, b=b):
                copy(g + 2, b).start()

        return carry

    zf = jnp.zeros((16,), jnp.float32)
    acc = lax.fori_loop(0, GATHERS // 2, pair_body,
                        (zf, zf, jnp.zeros((16,), jnp.uint32)))

    part_v[0] = acc[0]
    part_v[1] = acc[1]
    part_v[2] = acc[2].astype(jnp.float32)
    pltpu.sync_copy(part_v, out_hbm.at[c * 16 + s])


_sc_call = pl.kernel(
    _sc_body,
    out_type=jax.ShapeDtypeStruct((32, 3, 16), jnp.float32),
    mesh=plsc.VectorSubcoreMesh(core_axis_name="c", subcore_axis_name="s"),
    compiler_params=pltpu.CompilerParams(needs_layout_passes=False),
    scratch_types=[
        pltpu.VMEM((16,), jnp.int32),
        pltpu.VMEM((8, 4), jnp.int32),
        pltpu.VMEM((WORDROWS, 16), jnp.uint32),
        pltpu.VMEM((4, 8, 512), jnp.float32),
        pltpu.VMEM((4, 8, 512), jnp.float32),
        pltpu.VMEM((3, 16), jnp.float32),
        pltpu.SemaphoreType.DMA,
        pltpu.SemaphoreType.DMA,
    ],
)


def _finish_body(p_ref, cls_ref, out_ref):
    x = p_ref[0:16] + p_ref[16:32]
    s = x[:, 0, :].sum(-1, keepdims=True)
    s2 = x[:, 1, :].sum(-1, keepdims=True)
    c = x[:, 2, :].sum(-1, keepdims=True)
    safe = jnp.maximum(c, 1.0)
    means = jnp.where(c > 0, s / safe, 0.0)
    var = jnp.where(c > 0, s2 / safe - means * means, 0.0)
    row = jax.lax.broadcasted_iota(jnp.int32, (K, K), 0)
    col = jax.lax.broadcasted_iota(jnp.int32, (K, K), 1)
    eye = (row == col).astype(jnp.float32)
    mcol = jnp.broadcast_to(means, (K, K))
    mrow = (mcol * eye).sum(axis=0, keepdims=True)
    diff = mcol - mrow
    cls = cls_ref[...].astype(jnp.float32)
    ccol = (jnp.broadcast_to(cls, (K, K)) * eye).sum(axis=-1, keepdims=True)
    same = (jnp.broadcast_to(ccol, (K, K)) == cls).astype(jnp.float32)
    triu = (col > row).astype(jnp.float32)
    inter = jnp.sum(jnp.maximum(1.0 - diff * diff, 0.0) * same * triu)
    reg = jnp.mean(means * means)
    intra = jnp.mean(var)
    out_ref[...] = jnp.reshape(inter + reg + intra, (1, 1))


def _pack_body(mask_ref, out_ref):
    col = lax.broadcasted_iota(jnp.int32, (512, 32), 0)
    c2 = lax.broadcasted_iota(jnp.int32, (512, 32), 1)
    j = col // 16
    match = jnp.logical_and(col % 16 == c2 % 16, j // 16 == c2 // 16)
    powf = lax.bitcast_convert_type((j % 16 + 127) << 23, jnp.float32)
    proj = jnp.where(match, powf, 0.0)
    for i in range(8):
        m = mask_ref[i].astype(jnp.float32)
        r = jnp.dot(m, proj, preferred_element_type=jnp.float32)
        w = r[:, :16].astype(jnp.int32) | (r[:, 16:].astype(jnp.int32) << 16)
        out_ref[i * 512:(i + 1) * 512] = lax.bitcast_convert_type(w, jnp.uint32)


def _pack_mask(gt_objmask):
    return pl.pallas_call(
        _pack_body,
        grid=(2,),
        in_specs=[pl.BlockSpec((8, 512, 512), lambda b: (b, 0, 0))],
        out_specs=pl.BlockSpec((4096, 16), lambda b: (b, 0)),
        out_shape=jax.ShapeDtypeStruct((8192, 16), jnp.uint32),
    )(gt_objmask.view(jnp.uint8))


def kernel(pred_emb, gt_objmask, gt_classes):
    cls = gt_classes.astype(jnp.int32)
    pred_view = pred_emb.reshape(80 * NSLAB, 8, 512)
    words = _pack_mask(gt_objmask)
    partials = _sc_call(pred_view, words, cls)
    loss = pl.pallas_call(
        _finish_body,
        out_shape=jax.ShapeDtypeStruct((1, 1), jnp.float32),
    )(partials, cls[None, :])
    return loss.reshape(1)


# speedup vs baseline: 1.5806x; 1.0026x over previous
import jax
import jax.numpy as jnp
from jax import lax
from jax.experimental import pallas as pl
from jax.experimental.pallas import tpu as pltpu
from jax.experimental.pallas import tpu_sc as plsc

K = 16
NSLAB = 64
GATHERS = 8
WORDROWS = 256


def _sc_body(pred_hbm, words_hbm, cls_hbm, out_hbm,
             cls_v, idx_v, mask_v, eb, part_v, sems):
    c = lax.axis_index("c")
    s = lax.axis_index("s")
    wid = s * 2 + c

    pltpu.sync_copy(cls_hbm, cls_v)
    pltpu.sync_copy(words_hbm.at[pl.ds(wid * WORDROWS, WORDROWS)], mask_v)

    lane = lax.iota(jnp.int32, 16)
    cls_splat = plsc.load_gather(cls_v, [jnp.full((16,), s, jnp.int32)])
    base = cls_splat * NSLAB + c * (NSLAB // 2) + lane
    plsc.store_scatter(idx_v, [lane // 4, lane % 4], base)
    plsc.store_scatter(idx_v, [lane // 4 + 4, lane % 4], base + 16)

    def copy(g):
        par = lax.rem(g, 2)
        return pltpu.make_async_copy(
            pred_hbm.at[idx_v.at[g]],
            eb.at[pl.ds(par * 4, 4)],
            sems.at[par],
        )

    copy(0).start()
    copy(1).start()

    def gather_body(g, carry):
        copy(g).wait()
        half = lax.rem(g, 2) * 4

        def row_body(t, carry):
            sA, sB, cn = carry
            w = mask_v[g * 32 + t]
            i2 = half + t // 8
            r = t % 8
            for j in range(32):
                e = eb[i2, r, pl.ds(j * 16, 16)]
                msk = (w & jnp.uint32(1 << j)) != jnp.uint32(0)
                em = jnp.where(msk, e, 0.0)
                sA = sA + em
                sB = sB + em * em
            x = w - ((w >> jnp.uint32(1)) & jnp.uint32(0x55555555))
            x = (x & jnp.uint32(0x33333333)) + ((x >> jnp.uint32(2)) & jnp.uint32(0x33333333))
            x = (x + (x >> jnp.uint32(4))) & jnp.uint32(0x0F0F0F0F)
            cn = cn + ((x * jnp.uint32(0x01010101)) >> jnp.uint32(24))
            return (sA, sB, cn)

        carry = lax.fori_loop(0, 32, row_body, carry)

        @pl.when(g + 2 < GATHERS)
        def _next():
            copy(g + 2).start()

        return carry

    zf = jnp.zeros((16,), jnp.float32)
    acc = lax.fori_loop(0, GATHERS, gather_body,
                        (zf, zf, jnp.zeros((16,), jnp.uint32)))

    part_v[0] = acc[0]
    part_v[1] = acc[1]
    part_v[2] = acc[2].astype(jnp.float32)
    pltpu.sync_copy(part_v, out_hbm.at[c * 16 + s])


_sc_call = pl.kernel(
    _sc_body,
    out_type=jax.ShapeDtypeStruct((32, 3, 16), jnp.float32),
    mesh=plsc.VectorSubcoreMesh(core_axis_name="c", subcore_axis_name="s"),
    compiler_params=pltpu.CompilerParams(needs_layout_passes=False),
    scratch_types=[
        pltpu.VMEM((16,), jnp.int32),
        pltpu.VMEM((8, 4), jnp.int32),
        pltpu.VMEM((WORDROWS, 16), jnp.uint32),
        pltpu.VMEM((8, 8, 512), jnp.float32),
        pltpu.VMEM((3, 16), jnp.float32),
        pltpu.SemaphoreType.DMA((2,)),
    ],
)


def _finish_body(p_ref, cls_ref, out_ref):
    x = p_ref[0:16] + p_ref[16:32]
    s = x[:, 0, :].sum(-1, keepdims=True)
    s2 = x[:, 1, :].sum(-1, keepdims=True)
    c = x[:, 2, :].sum(-1, keepdims=True)
    safe = jnp.maximum(c, 1.0)
    means = jnp.where(c > 0, s / safe, 0.0)
    var = jnp.where(c > 0, s2 / safe - means * means, 0.0)
    row = jax.lax.broadcasted_iota(jnp.int32, (K, K), 0)
    col = jax.lax.broadcasted_iota(jnp.int32, (K, K), 1)
    eye = (row == col).astype(jnp.float32)
    mcol = jnp.broadcast_to(means, (K, K))
    mrow = (mcol * eye).sum(axis=0, keepdims=True)
    diff = mcol - mrow
    cls = cls_ref[...].astype(jnp.float32)
    ccol = (jnp.broadcast_to(cls, (K, K)) * eye).sum(axis=-1, keepdims=True)
    same = (jnp.broadcast_to(ccol, (K, K)) == cls).astype(jnp.float32)
    triu = (col > row).astype(jnp.float32)
    inter = jnp.sum(jnp.maximum(1.0 - diff * diff, 0.0) * same * triu)
    reg = jnp.mean(means * means)
    intra = jnp.mean(var)
    out_ref[...] = jnp.reshape(inter + reg + intra, (1, 1))


def _pack_body(mask_ref, out_ref):
    col = lax.broadcasted_iota(jnp.int32, (512, 32), 0)
    c2 = lax.broadcasted_iota(jnp.int32, (512, 32), 1)
    j = col // 16
    match = jnp.logical_and(col % 16 == c2 % 16, j // 16 == c2 // 16)
    powf = lax.bitcast_convert_type((j % 16 + 127) << 23, jnp.float32)
    proj = jnp.where(match, powf, 0.0)
    for i in range(8):
        m = mask_ref[i].astype(jnp.float32)
        r = jnp.dot(m, proj, preferred_element_type=jnp.float32)
        w = r[:, :16].astype(jnp.int32) | (r[:, 16:].astype(jnp.int32) << 16)
        out_ref[i * 512:(i + 1) * 512] = lax.bitcast_convert_type(w, jnp.uint32)


def _pack_mask(gt_objmask):
    return pl.pallas_call(
        _pack_body,
        grid=(2,),
        in_specs=[pl.BlockSpec((8, 512, 512), lambda b: (b, 0, 0))],
        out_specs=pl.BlockSpec((4096, 16), lambda b: (b, 0)),
        out_shape=jax.ShapeDtypeStruct((8192, 16), jnp.uint32),
    )(gt_objmask.view(jnp.uint8))


def kernel(pred_emb, gt_objmask, gt_classes):
    cls = gt_classes.astype(jnp.int32)
    pred_view = pred_emb.reshape(80 * NSLAB, 8, 512)
    words = _pack_mask(gt_objmask)
    partials = _sc_call(pred_view, words, cls)
    loss = pl.pallas_call(
        _finish_body,
        out_shape=jax.ShapeDtypeStruct((1, 1), jnp.float32),
    )(partials, cls[None, :])
    return loss.reshape(1)


# speedup vs baseline: 1.8315x; 1.1588x over previous
import jax
import jax.numpy as jnp
from jax import lax
from jax.experimental import pallas as pl
from jax.experimental.pallas import tpu as pltpu
from jax.experimental.pallas import tpu_sc as plsc

K = 16
NSC = 8
NSLAB = 64
GATHERS = 4
WORDROWS = 128


def _sc_body(pred_hbm, words_hbm, cls_hbm, out_hbm,
             cls_v, idx_v, mask_v, eb, part_v, sems):
    c = lax.axis_index("c")
    s = lax.axis_index("s")
    w = s * 2 + c
    p = w // 4
    q = w % 4

    pltpu.sync_copy(cls_hbm, cls_v)
    pltpu.sync_copy(words_hbm.at[pl.ds(w * WORDROWS, WORDROWS)], mask_v)

    lane = lax.iota(jnp.int32, 16)
    cls_splat = plsc.load_gather(cls_v, [jnp.full((16,), p, jnp.int32)])
    slab_ids = cls_splat * NSLAB + q * 16 + lane
    plsc.store_scatter(idx_v, [lane // 4, lane % 4], slab_ids)

    def copy(g):
        par = lax.rem(g, 2)
        return pltpu.make_async_copy(
            pred_hbm.at[idx_v.at[g]],
            eb.at[pl.ds(par * 4, 4)],
            sems.at[par],
        )

    copy(0).start()
    copy(1).start()

    def gather_body(g, carry):
        copy(g).wait()
        half = lax.rem(g, 2) * 4

        def row_body(t, carry):
            sA, sB, cn = carry
            wv = mask_v[g * 32 + t]
            i2 = half + t // 8
            r = t % 8
            for j in range(32):
                e = eb[i2, r, pl.ds(j * 16, 16)]
                msk = (wv & jnp.uint32(1 << j)) != jnp.uint32(0)
                em = jnp.where(msk, e, 0.0)
                sA = sA + em
                sB = sB + em * em
            x = wv - ((wv >> jnp.uint32(1)) & jnp.uint32(0x55555555))
            x = (x & jnp.uint32(0x33333333)) + ((x >> jnp.uint32(2)) & jnp.uint32(0x33333333))
            x = (x + (x >> jnp.uint32(4))) & jnp.uint32(0x0F0F0F0F)
            cn = cn + ((x * jnp.uint32(0x01010101)) >> jnp.uint32(24))
            return (sA, sB, cn)

        carry = lax.fori_loop(0, 32, row_body, carry)

        @pl.when(g + 2 < GATHERS)
        def _next():
            copy(g + 2).start()

        return carry

    zf = jnp.zeros((16,), jnp.float32)
    acc = lax.fori_loop(0, GATHERS, gather_body,
                        (zf, zf, jnp.zeros((16,), jnp.uint32)))

    part_v[0] = acc[0]
    part_v[1] = acc[1]
    part_v[2] = acc[2].astype(jnp.float32)
    pltpu.sync_copy(part_v, out_hbm.at[q * NSC + p])


_sc_call = pl.kernel(
    _sc_body,
    out_type=jax.ShapeDtypeStruct((32, 3, 16), jnp.float32),
    mesh=plsc.VectorSubcoreMesh(core_axis_name="c", subcore_axis_name="s"),
    compiler_params=pltpu.CompilerParams(needs_layout_passes=False),
    scratch_types=[
        pltpu.VMEM((16,), jnp.int32),
        pltpu.VMEM((4, 4), jnp.int32),
        pltpu.VMEM((WORDROWS, 16), jnp.uint32),
        pltpu.VMEM((8, 8, 512), jnp.float32),
        pltpu.VMEM((3, 16), jnp.float32),
        pltpu.SemaphoreType.DMA((2,)),
    ],
)


def _tc_reduce_body(classes_smem, emb_ref, mask_ref, out_ref):
    e = emb_ref[0]
    m = mask_ref[0].astype(jnp.float32)
    em = e * m

    def fold(x):
        return x.sum(axis=0).reshape(4, 128).sum(axis=0)

    out_ref[0, 0] = fold(em)
    out_ref[0, 1] = fold(em * e)
    out_ref[0, 2] = fold(m)


def _tc_partials(cls, pred_emb, mask_u8):
    grid_spec = pltpu.PrefetchScalarGridSpec(
        num_scalar_prefetch=1,
        grid=(K - NSC,),
        in_specs=[
            pl.BlockSpec((1, 512, 512), lambda k, classes: (classes[k + NSC], 0, 0)),
            pl.BlockSpec((1, 512, 512), lambda k, classes: (k + NSC, 0, 0)),
        ],
        out_specs=pl.BlockSpec((1, 3, 128), lambda k, classes: (k, 0, 0)),
    )
    return pl.pallas_call(
        _tc_reduce_body,
        grid_spec=grid_spec,
        out_shape=jax.ShapeDtypeStruct((K - NSC, 3, 128), jnp.float32),
    )(cls, pred_emb, mask_u8)


def _finish_body(psc_ref, ptc_ref, cls_ref, out_ref):
    x = psc_ref[0:8] + psc_ref[8:16] + psc_ref[16:24] + psc_ref[24:32]
    s_sc = x[:, 0, :].sum(-1, keepdims=True)
    s2_sc = x[:, 1, :].sum(-1, keepdims=True)
    c_sc = x[:, 2, :].sum(-1, keepdims=True)
    pt = ptc_ref[...]
    s = jnp.concatenate([s_sc, pt[:, 0, :].sum(-1, keepdims=True)], axis=0)
    s2 = jnp.concatenate([s2_sc, pt[:, 1, :].sum(-1, keepdims=True)], axis=0)
    c = jnp.concatenate([c_sc, pt[:, 2, :].sum(-1, keepdims=True)], axis=0)
    safe = jnp.maximum(c, 1.0)
    means = jnp.where(c > 0, s / safe, 0.0)
    var = jnp.where(c > 0, s2 / safe - means * means, 0.0)
    row = jax.lax.broadcasted_iota(jnp.int32, (K, K), 0)
    col = jax.lax.broadcasted_iota(jnp.int32, (K, K), 1)
    eye = (row == col).astype(jnp.float32)
    mcol = jnp.broadcast_to(means, (K, K))
    mrow = (mcol * eye).sum(axis=0, keepdims=True)
    diff = mcol - mrow
    cls = cls_ref[...].astype(jnp.float32)
    ccol = (jnp.broadcast_to(cls, (K, K)) * eye).sum(axis=-1, keepdims=True)
    same = (jnp.broadcast_to(ccol, (K, K)) == cls).astype(jnp.float32)
    triu = (col > row).astype(jnp.float32)
    inter = jnp.sum(jnp.maximum(1.0 - diff * diff, 0.0) * same * triu)
    reg = jnp.mean(means * means)
    intra = jnp.mean(var)
    out_ref[...] = jnp.reshape(inter + reg + intra, (1, 1))


def _pack_body(mask_ref, out_ref):
    col = lax.broadcasted_iota(jnp.int32, (512, 32), 0)
    c2 = lax.broadcasted_iota(jnp.int32, (512, 32), 1)
    j = col // 16
    match = jnp.logical_and(col % 16 == c2 % 16, j // 16 == c2 // 16)
    powf = lax.bitcast_convert_type((j % 16 + 127) << 23, jnp.float32)
    proj = jnp.where(match, powf, 0.0)
    for i in range(NSC):
        m = mask_ref[i].astype(jnp.float32)
        r = jnp.dot(m, proj, preferred_element_type=jnp.float32)
        w = r[:, :16].astype(jnp.int32) | (r[:, 16:].astype(jnp.int32) << 16)
        out_ref[i * 512:(i + 1) * 512] = lax.bitcast_convert_type(w, jnp.uint32)


def _pack_mask(mask_u8):
    return pl.pallas_call(
        _pack_body,
        grid=(1,),
        in_specs=[pl.BlockSpec((NSC, 512, 512), lambda b: (0, 0, 0))],
        out_specs=pl.BlockSpec((NSC * 512, 16), lambda b: (0, 0)),
        out_shape=jax.ShapeDtypeStruct((NSC * 512, 16), jnp.uint32),
    )(mask_u8)


def kernel(pred_emb, gt_objmask, gt_classes):
    cls = gt_classes.astype(jnp.int32)
    pred_view = pred_emb.reshape(80 * NSLAB, 8, 512)
    mask_u8 = gt_objmask.view(jnp.uint8)
    words = _pack_mask(mask_u8)
    partials_sc = _sc_call(pred_view, words, cls)
    partials_tc = _tc_partials(cls, pred_emb, mask_u8)
    loss = pl.pallas_call(
        _finish_body,
        out_shape=jax.ShapeDtypeStruct((1, 1), jnp.float32),
    )(partials_sc, partials_tc, cls[None, :])
    return loss.reshape(1)


# speedup vs baseline: 1.8316x; 1.0000x over previous
import jax
import jax.numpy as jnp
from jax import lax
from jax.experimental import pallas as pl
from jax.experimental.pallas import tpu as pltpu
from jax.experimental.pallas import tpu_sc as plsc

K = 16
NSC = 8
NSLAB = 64
GATHERS = 4
WORDROWS = 128


def _sc_body(pred_hbm, words_hbm, cls_hbm, out_hbm,
             cls_v, idx_v, mask_v, eb, part_v, sems):
    c = lax.axis_index("c")
    s = lax.axis_index("s")
    w = s * 2 + c
    p = w // 4
    q = w % 4

    pltpu.sync_copy(cls_hbm, cls_v)
    pltpu.sync_copy(words_hbm.at[pl.ds(w * WORDROWS, WORDROWS)], mask_v)

    lane = lax.iota(jnp.int32, 16)
    cls_splat = plsc.load_gather(cls_v, [jnp.full((16,), p, jnp.int32)])
    slab_ids = cls_splat * NSLAB + q * 16 + lane
    plsc.store_scatter(idx_v, [lane // 4, lane % 4], slab_ids)

    def copy(g):
        par = lax.rem(g, 2)
        return pltpu.make_async_copy(
            pred_hbm.at[idx_v.at[g]],
            eb.at[pl.ds(par * 4, 4)],
            sems.at[par],
        )

    copy(0).start()
    copy(1).start()

    def gather_body(g, carry):
        copy(g).wait()
        half = lax.rem(g, 2) * 4

        def row_body(t, carry):
            sA, sB, cn = carry
            wv = mask_v[g * 32 + t]
            i2 = half + t // 8
            r = t % 8
            for j in range(32):
                e = eb[i2, r, pl.ds(j * 16, 16)]
                msk = (wv & jnp.uint32(1 << j)) != jnp.uint32(0)
                em = jnp.where(msk, e, 0.0)
                sA = sA + em
                sB = sB + em * em
            x = wv - ((wv >> jnp.uint32(1)) & jnp.uint32(0x55555555))
            x = (x & jnp.uint32(0x33333333)) + ((x >> jnp.uint32(2)) & jnp.uint32(0x33333333))
            x = (x + (x >> jnp.uint32(4))) & jnp.uint32(0x0F0F0F0F)
            cn = cn + ((x * jnp.uint32(0x01010101)) >> jnp.uint32(24))
            return (sA, sB, cn)

        carry = lax.fori_loop(0, 32, row_body, carry)

        @pl.when(g + 2 < GATHERS)
        def _next():
            copy(g + 2).start()

        return carry

    zf = jnp.zeros((16,), jnp.float32)
    acc = lax.fori_loop(0, GATHERS, gather_body,
                        (zf, zf, jnp.zeros((16,), jnp.uint32)))

    part_v[0] = acc[0]
    part_v[1] = acc[1]
    part_v[2] = acc[2].astype(jnp.float32)
    pltpu.sync_copy(part_v, out_hbm.at[q * NSC + p])


_sc_call = pl.kernel(
    _sc_body,
    out_type=jax.ShapeDtypeStruct((32, 3, 16), jnp.float32),
    mesh=plsc.VectorSubcoreMesh(core_axis_name="c", subcore_axis_name="s"),
    compiler_params=pltpu.CompilerParams(
        needs_layout_passes=False,
        disable_bounds_checks=True,
        disable_semaphore_checks=True,
        skip_device_barrier=True,
    ),
    scratch_types=[
        pltpu.VMEM((16,), jnp.int32),
        pltpu.VMEM((4, 4), jnp.int32),
        pltpu.VMEM((WORDROWS, 16), jnp.uint32),
        pltpu.VMEM((8, 8, 512), jnp.float32),
        pltpu.VMEM((3, 16), jnp.float32),
        pltpu.SemaphoreType.DMA((2,)),
    ],
)


def _tc_reduce_body(classes_smem, emb_ref, mask_ref, out_ref):
    e = emb_ref[0]
    m = mask_ref[0].astype(jnp.float32)
    em = e * m

    def fold(x):
        return x.sum(axis=0).reshape(4, 128).sum(axis=0)

    out_ref[0, 0] = fold(em)
    out_ref[0, 1] = fold(em * e)
    out_ref[0, 2] = fold(m)


def _tc_partials(cls, pred_emb, mask_u8):
    grid_spec = pltpu.PrefetchScalarGridSpec(
        num_scalar_prefetch=1,
        grid=(K - NSC,),
        in_specs=[
            pl.BlockSpec((1, 512, 512), lambda k, classes: (classes[k + NSC], 0, 0)),
            pl.BlockSpec((1, 512, 512), lambda k, classes: (k + NSC, 0, 0)),
        ],
        out_specs=pl.BlockSpec((1, 3, 128), lambda k, classes: (k, 0, 0)),
    )
    return pl.pallas_call(
        _tc_reduce_body,
        grid_spec=grid_spec,
        out_shape=jax.ShapeDtypeStruct((K - NSC, 3, 128), jnp.float32),
    )(cls, pred_emb, mask_u8)


def _finish_body(psc_ref, ptc_ref, cls_ref, out_ref):
    x = psc_ref[0:8] + psc_ref[8:16] + psc_ref[16:24] + psc_ref[24:32]
    s_sc = x[:, 0, :].sum(-1, keepdims=True)
    s2_sc = x[:, 1, :].sum(-1, keepdims=True)
    c_sc = x[:, 2, :].sum(-1, keepdims=True)
    pt = ptc_ref[...]
    s = jnp.concatenate([s_sc, pt[:, 0, :].sum(-1, keepdims=True)], axis=0)
    s2 = jnp.concatenate([s2_sc, pt[:, 1, :].sum(-1, keepdims=True)], axis=0)
    c = jnp.concatenate([c_sc, pt[:, 2, :].sum(-1, keepdims=True)], axis=0)
    safe = jnp.maximum(c, 1.0)
    means = jnp.where(c > 0, s / safe, 0.0)
    var = jnp.where(c > 0, s2 / safe - means * means, 0.0)
    row = jax.lax.broadcasted_iota(jnp.int32, (K, K), 0)
    col = jax.lax.broadcasted_iota(jnp.int32, (K, K), 1)
    eye = (row == col).astype(jnp.float32)
    mcol = jnp.broadcast_to(means, (K, K))
    mrow = (mcol * eye).sum(axis=0, keepdims=True)
    diff = mcol - mrow
    cls = cls_ref[...].astype(jnp.float32)
    ccol = (jnp.broadcast_to(cls, (K, K)) * eye).sum(axis=-1, keepdims=True)
    same = (jnp.broadcast_to(ccol, (K, K)) == cls).astype(jnp.float32)
    triu = (col > row).astype(jnp.float32)
    inter = jnp.sum(jnp.maximum(1.0 - diff * diff, 0.0) * same * triu)
    reg = jnp.mean(means * means)
    intra = jnp.mean(var)
    out_ref[...] = jnp.reshape(inter + reg + intra, (1, 1))


def _pack_body(mask_ref, out_ref):
    col = lax.broadcasted_iota(jnp.int32, (512, 32), 0)
    c2 = lax.broadcasted_iota(jnp.int32, (512, 32), 1)
    j = col // 16
    match = jnp.logical_and(col % 16 == c2 % 16, j // 16 == c2 // 16)
    powf = lax.bitcast_convert_type((j % 16 + 127) << 23, jnp.float32)
    proj = jnp.where(match, powf, 0.0)
    for i in range(NSC):
        m = mask_ref[i].astype(jnp.float32)
        r = jnp.dot(m, proj, preferred_element_type=jnp.float32)
        w = r[:, :16].astype(jnp.int32) | (r[:, 16:].astype(jnp.int32) << 16)
        out_ref[i * 512:(i + 1) * 512] = lax.bitcast_convert_type(w, jnp.uint32)


def _pack_mask(mask_u8):
    return pl.pallas_call(
        _pack_body,
        grid=(1,),
        in_specs=[pl.BlockSpec((NSC, 512, 512), lambda b: (0, 0, 0))],
        out_specs=pl.BlockSpec((NSC * 512, 16), lambda b: (0, 0)),
        out_shape=jax.ShapeDtypeStruct((NSC * 512, 16), jnp.uint32),
    )(mask_u8)


def kernel(pred_emb, gt_objmask, gt_classes):
    cls = gt_classes.astype(jnp.int32)
    pred_view = pred_emb.reshape(80 * NSLAB, 8, 512)
    mask_u8 = gt_objmask.view(jnp.uint8)
    words = _pack_mask(mask_u8)
    partials_sc = _sc_call(pred_view, words, cls)
    partials_tc = _tc_partials(cls, pred_emb, mask_u8)
    loss = pl.pallas_call(
        _finish_body,
        out_shape=jax.ShapeDtypeStruct((1, 1), jnp.float32),
    )(partials_sc, partials_tc, cls[None, :])
    return loss.reshape(1)


# speedup vs baseline: 1.8387x; 1.0039x over previous
import jax
import jax.numpy as jnp
from jax import lax
from jax.experimental import pallas as pl
from jax.experimental.pallas import tpu as pltpu
from jax.experimental.pallas import tpu_sc as plsc

K = 16
NSC = 8
NSLAB = 64
GATHERS = 4
WORDROWS = 128


def _sc_body(pred_hbm, words_hbm, cls_hbm, out_hbm,
             cls_v, idx_v, mask_v, eb, part_v, sems):
    c = lax.axis_index("c")
    s = lax.axis_index("s")
    w = s * 2 + c
    p = w // 4
    q = w % 4

    pltpu.sync_copy(cls_hbm, cls_v)
    pltpu.sync_copy(words_hbm.at[pl.ds(w * WORDROWS, WORDROWS)], mask_v)

    lane = lax.iota(jnp.int32, 16)
    cls_splat = plsc.load_gather(cls_v, [jnp.full((16,), p, jnp.int32)])
    slab_ids = cls_splat * NSLAB + q * 16 + lane
    plsc.store_scatter(idx_v, [lane // 4, lane % 4], slab_ids)

    def copy(g):
        par = lax.rem(g, 2)
        return pltpu.make_async_copy(
            pred_hbm.at[idx_v.at[g]],
            eb.at[pl.ds(par * 4, 4)],
            sems.at[par],
        )

    copy(0).start()
    copy(1).start()

    def gather_body(g, carry):
        copy(g).wait()
        half = lax.rem(g, 2) * 4

        def row_body(t, carry):
            sA, sB, cn = carry
            wv = mask_v[g * 32 + t]
            i2 = half + t // 8
            r = t % 8
            for j in range(32):
                e = eb[i2, r, pl.ds(j * 16, 16)]
                msk = (wv & jnp.uint32(1 << j)) != jnp.uint32(0)
                em = jnp.where(msk, e, 0.0)
                sA = sA + em
                sB = sB + em * em
            x = wv - ((wv >> jnp.uint32(1)) & jnp.uint32(0x55555555))
            x = (x & jnp.uint32(0x33333333)) + ((x >> jnp.uint32(2)) & jnp.uint32(0x33333333))
            x = (x + (x >> jnp.uint32(4))) & jnp.uint32(0x0F0F0F0F)
            cn = cn + ((x * jnp.uint32(0x01010101)) >> jnp.uint32(24))
            return (sA, sB, cn)

        carry = lax.fori_loop(0, 32, row_body, carry)

        @pl.when(g + 2 < GATHERS)
        def _next():
            copy(g + 2).start()

        return carry

    zf = jnp.zeros((16,), jnp.float32)
    acc = lax.fori_loop(0, GATHERS, gather_body,
                        (zf, zf, jnp.zeros((16,), jnp.uint32)))

    part_v[0] = acc[0]
    part_v[1] = acc[1]
    part_v[2] = acc[2].astype(jnp.float32)
    pltpu.sync_copy(part_v, out_hbm.at[q * NSC + p])


_sc_call = pl.kernel(
    _sc_body,
    out_type=jax.ShapeDtypeStruct((32, 3, 16), jnp.float32),
    mesh=plsc.VectorSubcoreMesh(core_axis_name="c", subcore_axis_name="s"),
    compiler_params=pltpu.CompilerParams(
        needs_layout_passes=False,
        disable_bounds_checks=True,
        disable_semaphore_checks=True,
        skip_device_barrier=True,
    ),
    scratch_types=[
        pltpu.VMEM((16,), jnp.int32),
        pltpu.VMEM((4, 4), jnp.int32),
        pltpu.VMEM((WORDROWS, 16), jnp.uint32),
        pltpu.VMEM((8, 8, 512), jnp.float32),
        pltpu.VMEM((3, 16), jnp.float32),
        pltpu.SemaphoreType.DMA((2,)),
    ],
)


def _tc_reduce_body(classes_smem, emb_ref, mask_ref, out_ref):
    e = emb_ref[0]
    m = mask_ref[0].astype(jnp.float32)
    em = e * m

    def fold(x):
        return x.sum(axis=0).reshape(4, 128).sum(axis=0)

    out_ref[0, 0] = fold(em)
    out_ref[0, 1] = fold(em * e)
    out_ref[0, 2] = fold(m)


def _tc_partials(cls, pred_emb, mask_u8):
    grid_spec = pltpu.PrefetchScalarGridSpec(
        num_scalar_prefetch=1,
        grid=(K - NSC,),
        in_specs=[
            pl.BlockSpec((1, 512, 512), lambda k, classes: (classes[k + NSC], 0, 0)),
            pl.BlockSpec((1, 512, 512), lambda k, classes: (k, 0, 0)),
        ],
        out_specs=pl.BlockSpec((1, 3, 128), lambda k, classes: (k, 0, 0)),
    )
    return pl.pallas_call(
        _tc_reduce_body,
        grid_spec=grid_spec,
        out_shape=jax.ShapeDtypeStruct((K - NSC, 3, 128), jnp.float32),
    )(cls, pred_emb, mask_u8)


def _finish_body(psc_ref, ptc_ref, cls_ref, out_ref):
    x = psc_ref[0:8] + psc_ref[8:16] + psc_ref[16:24] + psc_ref[24:32]
    s_sc = x[:, 0, :].sum(-1, keepdims=True)
    s2_sc = x[:, 1, :].sum(-1, keepdims=True)
    c_sc = x[:, 2, :].sum(-1, keepdims=True)
    pt = ptc_ref[...]
    s = jnp.concatenate([s_sc, pt[:, 0, :].sum(-1, keepdims=True)], axis=0)
    s2 = jnp.concatenate([s2_sc, pt[:, 1, :].sum(-1, keepdims=True)], axis=0)
    c = jnp.concatenate([c_sc, pt[:, 2, :].sum(-1, keepdims=True)], axis=0)
    safe = jnp.maximum(c, 1.0)
    means = jnp.where(c > 0, s / safe, 0.0)
    var = jnp.where(c > 0, s2 / safe - means * means, 0.0)
    row = jax.lax.broadcasted_iota(jnp.int32, (K, K), 0)
    col = jax.lax.broadcasted_iota(jnp.int32, (K, K), 1)
    eye = (row == col).astype(jnp.float32)
    mcol = jnp.broadcast_to(means, (K, K))
    mrow = (mcol * eye).sum(axis=0, keepdims=True)
    diff = mcol - mrow
    cls = cls_ref[...].astype(jnp.float32)
    ccol = (jnp.broadcast_to(cls, (K, K)) * eye).sum(axis=-1, keepdims=True)
    same = (jnp.broadcast_to(ccol, (K, K)) == cls).astype(jnp.float32)
    triu = (col > row).astype(jnp.float32)
    inter = jnp.sum(jnp.maximum(1.0 - diff * diff, 0.0) * same * triu)
    reg = jnp.mean(means * means)
    intra = jnp.mean(var)
    out_ref[...] = jnp.reshape(inter + reg + intra, (1, 1))


def _pack_body(mask_ref, out_ref):
    col = lax.broadcasted_iota(jnp.int32, (512, 32), 0)
    c2 = lax.broadcasted_iota(jnp.int32, (512, 32), 1)
    j = col // 16
    match = jnp.logical_and(col % 16 == c2 % 16, j // 16 == c2 // 16)
    powf = lax.bitcast_convert_type((j % 16 + 127) << 23, jnp.float32)
    proj = jnp.where(match, powf, 0.0)
    for i in range(NSC):
        m = mask_ref[i].astype(jnp.float32)
        r = jnp.dot(m, proj, preferred_element_type=jnp.float32)
        w = r[:, :16].astype(jnp.int32) | (r[:, 16:].astype(jnp.int32) << 16)
        out_ref[i * 512:(i + 1) * 512] = lax.bitcast_convert_type(w, jnp.uint32)


def _pack_mask(mask_u8):
    return pl.pallas_call(
        _pack_body,
        grid=(1,),
        in_specs=[pl.BlockSpec((NSC, 512, 512), lambda b: (0, 0, 0))],
        out_specs=pl.BlockSpec((NSC * 512, 16), lambda b: (0, 0)),
        out_shape=jax.ShapeDtypeStruct((NSC * 512, 16), jnp.uint32),
    )(mask_u8)


def kernel(pred_emb, gt_objmask, gt_classes):
    cls = gt_classes.astype(jnp.int32)
    pred_view = pred_emb.reshape(80 * NSLAB, 8, 512)
    words = _pack_mask(gt_objmask[:NSC].view(jnp.uint8))
    partials_sc = _sc_call(pred_view, words, cls)
    partials_tc = _tc_partials(cls, pred_emb, gt_objmask[NSC:].view(jnp.uint8))
    loss = pl.pallas_call(
        _finish_body,
        out_shape=jax.ShapeDtypeStruct((1, 1), jnp.float32),
    )(partials_sc, partials_tc, cls[None, :])
    return loss.reshape(1)
